# baseline (speedup 1.0000x reference)
import functools

import jax
import jax.numpy as jnp
from jax import lax
from jax.experimental import pallas as pl
from jax.experimental.pallas import tpu as pltpu
from jax.experimental.pallas import tpu_sc as plsc

_DROP_LENGTH_LOW = 1000
_DROP_LENGTH_HIGH = 8000
_DROP_COUNT_LOW = 1
_DROP_COUNT_HIGH = 10
_SEED = 42

_B = 32
_T = 160000
_MAXD = _DROP_COUNT_HIGH
_NC = 2
_NS = 16
_CS = 20000
_NCH = _T // _CS
_NBUF = 4


def _interval_params(lengths):
    key = jax.random.key(_SEED)
    kp, kc, kl, ks = jax.random.split(key, 4)
    clean_length = (lengths * _T).astype(jnp.int32)
    drop_times = jax.random.randint(kc, (_B,), _DROP_COUNT_LOW, _DROP_COUNT_HIGH)
    chunk_len = jax.random.randint(
        kl, (_B, _MAXD), _DROP_LENGTH_LOW, _DROP_LENGTH_HIGH + 1)
    u = jax.random.uniform(ks, (_B, _MAXD))
    max_start = jnp.maximum(clean_length[:, None] - chunk_len, 1)
    start = (u * max_start.astype(jnp.float32)).astype(jnp.int32)
    valid = jnp.arange(_MAXD)[None, :] < drop_times[:, None]
    end = jnp.where(valid, start + chunk_len, start)
    s16 = jnp.zeros((_B, 16), jnp.int32).at[:, :_MAXD].set(start)
    e16 = jnp.zeros((_B, 16), jnp.int32).at[:, :_MAXD].set(end)
    return s16.reshape(-1), e16.reshape(-1)


def _zero_chunk(buf, c0, sv, ev, lane, zf):
    def interval_body(s, e):
        ls = jnp.clip(s - c0, 0, _CS)
        le = jnp.clip(e - c0, 0, _CS)
        le = jnp.maximum(le, ls)
        ls_a = jnp.minimum(((ls + 127) // 128) * 128, le)
        le_a = jnp.maximum((le // 128) * 128, ls_a)

        def rmw_body(g, _):
            base = g * 16
            pos = lane + base
            m = (pos >= ls) & (pos < le)
            buf[pl.ds(base, 16)] = jnp.where(m, 0.0, buf[pl.ds(base, 16)])
            return 0

        lax.fori_loop(ls // 16, (ls_a + 15) // 16, rmw_body, 0)

        def blk_body(i, _):
            base = ls_a + i * 128
            for k in range(8):
                buf[pl.ds(base + k * 16, 16)] = zf
            return 0

        lax.fori_loop(0, (le_a - ls_a) // 128, blk_body, 0)

        lax.fori_loop(le_a // 16, (le + 15) // 16, rmw_body, 0)

    for d in range(_MAXD):
        interval_body(sv[d], ev[d])


def _sc_body(w_hbm, s_hbm, e_hbm, out_hbm, sv_ref, ev_ref, bufs, gsems, ssems):
    cid = lax.axis_index("c")
    sid = lax.axis_index("s")
    wid = sid * _NC + cid
    rowbase = wid * _T

    pltpu.sync_copy(s_hbm.at[pl.ds(wid * 16, 16)], sv_ref)
    pltpu.sync_copy(e_hbm.at[pl.ds(wid * 16, 16)], ev_ref)
    sv = sv_ref[...]
    ev = ev_ref[...]
    lane = lax.broadcasted_iota(jnp.int32, (16,), 0)
    zf = jnp.zeros((16,), jnp.float32)

    gh = [None] * _NBUF
    sh = [None] * _NBUF
    for c in range(_NBUF):
        gh[c] = pltpu.async_copy(
            w_hbm.at[pl.ds(rowbase + c * _CS, _CS)], bufs[c], gsems[c])

    unwaited = []
    for c in range(_NCH):
        b = c % _NBUF
        gh[b].wait()
        _zero_chunk(bufs[b], c * _CS, sv, ev, lane, zf)
        sh[b] = pltpu.async_copy(
            bufs[b], out_hbm.at[pl.ds(rowbase + c * _CS, _CS)], ssems[b])
        unwaited.append(b)
        if c >= 1:
            pb = (c - 1) % _NBUF
            nxt = c - 1 + _NBUF
            if nxt < _NCH:
                sh[pb].wait()
                unwaited.remove(pb)
                gh[pb] = pltpu.async_copy(
                    w_hbm.at[pl.ds(rowbase + nxt * _CS, _CS)], bufs[pb],
                    gsems[pb])
    for b in unwaited:
        sh[b].wait()


def kernel(waveform, lengths):
    s_flat, e_flat = _interval_params(lengths)
    w_flat = waveform.reshape(-1)

    mesh = plsc.VectorSubcoreMesh(core_axis_name="c", subcore_axis_name="s")

    @functools.partial(
        pl.kernel,
        out_type=jax.ShapeDtypeStruct((_B * _T,), jnp.float32),
        mesh=mesh,
        scratch_types=[
            pltpu.VMEM((16,), jnp.int32),
            pltpu.VMEM((16,), jnp.int32),
            [pltpu.VMEM((_CS,), jnp.float32) for _ in range(_NBUF)],
            [pltpu.SemaphoreType.DMA for _ in range(_NBUF)],
            [pltpu.SemaphoreType.DMA for _ in range(_NBUF)],
        ],
    )
    def run(w_hbm, s_hbm, e_hbm, out_hbm, sv_ref, ev_ref, bufs, gsems, ssems):
        _sc_body(w_hbm, s_hbm, e_hbm, out_hbm, sv_ref, ev_ref, bufs, gsems,
                 ssems)

    out = run(w_flat, s_flat, e_flat)
    return out.reshape(_B, _T)
